# Initial kernel scaffold; baseline (speedup 1.0000x reference)
import jax
import jax.numpy as jnp
from jax.experimental import pallas as pl


def kernel(h, edge_index_mp0, edge_index_mp1, W0, al0, ar0, b0, W1, al1, ar1, b1, ln_g, ln_b, pW1, pb1, pw2):
    raise NotImplementedError("write your pallas kernel here")


# speedup vs baseline: 22.2353x; 22.2353x over previous
import functools
import jax
import jax.numpy as jnp
from jax import lax
from jax.experimental import pallas as pl
from jax.experimental.pallas import tpu as pltpu
from jax.experimental.pallas import tpu_sc as plsc

N = 50000
E = 800000
IN = 64
HEADS = 4
OUT = 16
SEM = 128

NPAD = 50176
QUARTER = NPAD // 4
ROWS_PER_SUB = QUARTER // 16
EPAD = 800768
E_PER_SUB = EPAD // 16
CHUNK = 128
NCHUNK = E_PER_SUB // CHUNK
TABW = 80


def _sc_edge_body(src_hbm, dst_hbm, tab_hbm, ert_hbm, zrows_hbm, out_hbm,
                  srcv, dstv, lidxv, rowsv, erv, msgv,
                  acc, sem1, sem2):
    c = lax.axis_index("c")
    s = lax.axis_index("s")

    for q in range(2):
        lo = (2 * c + q) * QUARTER

        pltpu.sync_copy(zrows_hbm, acc.at[pl.ds(s * ROWS_PER_SUB, ROWS_PER_SUB)])
        plsc.subcore_barrier()

        def chunk_body(t, carry):
            base = s * E_PER_SUB + t * CHUNK
            pltpu.sync_copy(src_hbm.at[pl.ds(base, CHUNK)], srcv)
            pltpu.sync_copy(dst_hbm.at[pl.ds(base, CHUNK)], dstv)
            pltpu.async_copy(tab_hbm.at[srcv], rowsv, sem1).wait()
            pltpu.async_copy(ert_hbm.at[dstv], erv, sem2).wait()

            for j in range(CHUNK // 16):
                d = dstv[pl.ds(j * 16, 16)]
                own = (d >= lo) & (d < lo + QUARTER)
                lidxv[pl.ds(j * 16, 16)] = jnp.where(own, d - lo, QUARTER)

            def edge_body(i, carry2):
                att = rowsv[i, pl.ds(64, 16)]
                erow = erv[i, pl.ds(0, 16)]
                v = att + erow
                e = jnp.maximum(v, 0.2 * v)
                ee = jnp.exp(e)
                msgv[i, pl.ds(64, 16)] = ee
                for h in range(HEADS):
                    msgv[i, pl.ds(h * 16, 16)] = rowsv[i, pl.ds(h * 16, 16)] * ee[h]
                return carry2

            lax.fori_loop(0, CHUNK, edge_body, 0)
            pltpu.sync_copy(msgv, acc.at[lidxv], add=True)
            return carry

        lax.fori_loop(0, NCHUNK, chunk_body, 0)
        plsc.subcore_barrier()

        pltpu.sync_copy(acc.at[pl.ds(s * ROWS_PER_SUB, ROWS_PER_SUB)],
                        out_hbm.at[pl.ds(lo + s * ROWS_PER_SUB, ROWS_PER_SUB)])
        plsc.subcore_barrier()


def _sc_edge(src, dst, tab, ert, zrows):
    mesh = plsc.VectorSubcoreMesh(core_axis_name="c", subcore_axis_name="s")
    k = functools.partial(
        pl.kernel, mesh=mesh,
        compiler_params=pltpu.CompilerParams(use_tc_tiling_on_sc=False),
        out_type=jax.ShapeDtypeStruct((NPAD, TABW), jnp.float32),
        scratch_types=[
            pltpu.VMEM((CHUNK,), jnp.int32),
            pltpu.VMEM((CHUNK,), jnp.int32),
            pltpu.VMEM((CHUNK,), jnp.int32),
            pltpu.VMEM((CHUNK, TABW), jnp.float32),
            pltpu.VMEM((CHUNK, 16), jnp.float32),
            pltpu.VMEM((CHUNK, TABW), jnp.float32),
            pltpu.VMEM_SHARED((QUARTER + 8, TABW), jnp.float32),
            pltpu.SemaphoreType.DMA,
            pltpu.SemaphoreType.DMA,
        ],
    )(_sc_edge_body)
    return k(src, dst, tab, ert, zrows)


def _pre_body(h_ref, w0_ref, a0_ref, b0m_ref, w1_ref, a1_ref, b1m_ref,
              g_ref, be_ref, tab0_ref, ert0_ref, tab1_ref, ert1_ref, ln_ref):
    h = h_ref[...]
    z12 = jnp.zeros((h.shape[0], 12), jnp.float32)
    f0 = jnp.dot(h, w0_ref[...], preferred_element_type=jnp.float32)
    el0 = jnp.dot(f0, a0_ref[...], preferred_element_type=jnp.float32)
    er0 = jnp.dot(f0, b0m_ref[...], preferred_element_type=jnp.float32)
    tab0_ref[...] = jnp.concatenate([f0, el0, z12], axis=1)
    ert0_ref[...] = jnp.concatenate([er0, z12], axis=1)
    f1 = jnp.dot(h, w1_ref[...], preferred_element_type=jnp.float32)
    el1 = jnp.dot(f1, a1_ref[...], preferred_element_type=jnp.float32)
    er1 = jnp.dot(f1, b1m_ref[...], preferred_element_type=jnp.float32)
    tab1_ref[...] = jnp.concatenate([f1, el1, z12], axis=1)
    ert1_ref[...] = jnp.concatenate([er1, z12], axis=1)
    mu = jnp.mean(h, axis=1, keepdims=True)
    var = jnp.mean((h - mu) ** 2, axis=1, keepdims=True)
    ln_ref[...] = (h - mu) / jnp.sqrt(var + 1e-5) * g_ref[...] + be_ref[...]


def _pre(h_pad, W0, A0, B0m, W1, A1, B1m, g2, b2):
    BLK = 1024
    G = NPAD // BLK
    full = lambda r, c: pl.BlockSpec((r, c), lambda i: (0, 0))
    blk = lambda c: pl.BlockSpec((BLK, c), lambda i: (i, 0))
    return pl.pallas_call(
        _pre_body,
        grid=(G,),
        in_specs=[blk(IN), full(IN, 64), full(64, HEADS), full(64, HEADS),
                  full(IN, 64), full(64, HEADS), full(64, HEADS),
                  full(1, IN), full(1, IN)],
        out_specs=[blk(TABW), blk(16), blk(TABW), blk(16), blk(IN)],
        out_shape=[
            jax.ShapeDtypeStruct((NPAD, TABW), jnp.float32),
            jax.ShapeDtypeStruct((NPAD, 16), jnp.float32),
            jax.ShapeDtypeStruct((NPAD, TABW), jnp.float32),
            jax.ShapeDtypeStruct((NPAD, 16), jnp.float32),
            jax.ShapeDtypeStruct((NPAD, IN), jnp.float32),
        ],
    )(h_pad, W0, A0, B0m, W1, A1, B1m, g2, b2)


def _postA_body(acc0_ref, acc1_ref, b0_ref, b1_ref, pw1_ref, pb1_ref,
                pw2_ref, z0_ref, z1_ref, part_ref):
    def head_norm(acc_ref):
        cols = []
        for h in range(HEADS):
            num = acc_ref[:, h * 16:(h + 1) * 16]
            den = jnp.maximum(acc_ref[:, 64 + h:65 + h], 1e-9)
            cols.append(num / den)
        return jnp.concatenate(cols, axis=1)

    def elu(x):
        return jnp.where(x > 0, x, jnp.exp(x) - 1.0)

    z0 = elu(head_norm(acc0_ref) + b0_ref[...])
    z1 = elu(head_norm(acc1_ref) + b1_ref[...])
    z0_ref[...] = z0
    z1_ref[...] = z1
    t0 = jnp.tanh(jnp.dot(z0, pw1_ref[...], preferred_element_type=jnp.float32)
                  + pb1_ref[...])
    t1 = jnp.tanh(jnp.dot(z1, pw1_ref[...], preferred_element_type=jnp.float32)
                  + pb1_ref[...])
    s0 = jnp.sum(jnp.dot(t0, pw2_ref[...], preferred_element_type=jnp.float32))
    s1 = jnp.sum(jnp.dot(t1, pw2_ref[...], preferred_element_type=jnp.float32))
    lanes = lax.broadcasted_iota(jnp.int32, (1, 128), 1)
    part_ref[...] = (jnp.where(lanes == 0, s0, 0.0)
                     + jnp.where(lanes == 1, s1, 0.0)).reshape(1, 1, 128)


def _postA(acc0, acc1, b0r, b1r, pW1, pb1r, pw2):
    BLK = 2000
    G = N // BLK
    full = lambda r, c: pl.BlockSpec((r, c), lambda i: (0, 0))
    blk = lambda c: pl.BlockSpec((BLK, c), lambda i: (i, 0))
    return pl.pallas_call(
        _postA_body,
        grid=(G,),
        in_specs=[blk(TABW), blk(TABW), full(1, 64), full(1, 64),
                  full(64, SEM), full(1, SEM), full(SEM, 1)],
        out_specs=[blk(64), blk(64),
                   pl.BlockSpec((1, 1, 128), lambda i: (i, 0, 0))],
        out_shape=[
            jax.ShapeDtypeStruct((N, 64), jnp.float32),
            jax.ShapeDtypeStruct((N, 64), jnp.float32),
            jax.ShapeDtypeStruct((G, 1, 128), jnp.float32),
        ],
    )(acc0, acc1, b0r, b1r, pW1, pb1r, pw2)


def _postB_body(ln_ref, z0_ref, z1_ref, beta0_ref, beta1_ref, out_ref):
    out_ref[...] = (ln_ref[...] + beta0_ref[...] * z0_ref[...]
                    + beta1_ref[...] * z1_ref[...])


def _postB(ln, z0, z1, beta0r, beta1r):
    BLK = 2000
    G = N // BLK
    full = lambda r, c: pl.BlockSpec((r, c), lambda i: (0, 0))
    blk = lambda c: pl.BlockSpec((BLK, c), lambda i: (i, 0))
    return pl.pallas_call(
        _postB_body,
        grid=(G,),
        in_specs=[blk(64), blk(64), blk(64), full(1, 64), full(1, 64)],
        out_specs=blk(64),
        out_shape=jax.ShapeDtypeStruct((N, 64), jnp.float32),
    )(ln, z0, z1, beta0r, beta1r)


def _headmat(a):
    m = jnp.zeros((IN, HEADS), jnp.float32)
    for h in range(HEADS):
        m = m.at[h * OUT:(h + 1) * OUT, h].set(a[0, h, :])
    return m


def kernel(h, edge_index_mp0, edge_index_mp1, W0, al0, ar0, b0, W1, al1, ar1,
           b1, ln_g, ln_b, pW1, pb1, pw2):
    h_pad = jnp.pad(h, ((0, NPAD - N), (0, 0)))
    pad = EPAD - E
    padi = jnp.full((pad,), NPAD - 1, jnp.int32)
    src0 = jnp.concatenate([edge_index_mp0[0], padi])
    dst0 = jnp.concatenate([edge_index_mp0[1], padi])
    src1 = jnp.concatenate([edge_index_mp1[0], padi])
    dst1 = jnp.concatenate([edge_index_mp1[1], padi])

    tab0, ert0, tab1, ert1, ln = _pre(
        h_pad, W0, _headmat(al0), _headmat(ar0), W1, _headmat(al1),
        _headmat(ar1), ln_g.reshape(1, IN), ln_b.reshape(1, IN))

    zrows = jnp.zeros((ROWS_PER_SUB, TABW), jnp.float32)
    acc0 = _sc_edge(src0, dst0, tab0, ert0, zrows)[:N]
    acc1 = _sc_edge(src1, dst1, tab1, ert1, zrows)[:N]

    z0, z1, parts = _postA(acc0, acc1, b0.reshape(1, 64), b1.reshape(1, 64),
                           pW1, pb1.reshape(1, SEM), pw2)
    w = parts.sum(axis=0).reshape(128)[:2] / N
    beta = jax.nn.softmax(w)
    beta0r = jnp.full((1, 64), beta[0])
    beta1r = jnp.full((1, 64), beta[1])
    return _postB(ln, z0, z1, beta0r, beta1r)
